# baseline (speedup 1.0000x reference)
import jax
import jax.numpy as jnp
from jax import lax
from jax.experimental import pallas as pl
from jax.experimental.pallas import tpu as pltpu
from jax.experimental.pallas import tpu_sc as plsc

B = 16384
D = 32
V = 1000000
NUM_CORES = 2
NUM_SUBCORES = 16
LANES = 16
NW = NUM_CORES * NUM_SUBCORES
BPW = B // NW
GCHUNK = 128
NDESC = BPW * D // GCHUNK


def _rsqrt_newton(t):
    i = lax.bitcast_convert_type(t, jnp.int32)
    y = lax.bitcast_convert_type(
        jnp.int32(0x5F3759DF) - lax.shift_right_logical(i, 1), jnp.float32
    )
    ht = t * jnp.float32(0.5)
    for _ in range(3):
        y = y * (jnp.float32(1.5) - ht * y * y)
    return y


def _hsum_all_lanes(v):
    lane = lax.iota(jnp.int32, LANES)
    for k in (8, 4, 2, 1):
        rot = lax.bitwise_and(lane + k, LANES - 1)
        v = v + jnp.take_along_axis(v, rot, axis=0)
    return v


def _sc_body(idx_hbm, tbl_hbm, out_hbm, idx_v, widx_v, buf, sem):
    wid = lax.axis_index("s") * NUM_CORES + lax.axis_index("c")
    base = wid * BPW

    pltpu.sync_copy(idx_hbm.at[pl.ds(base, BPW)], idx_v)

    jbase0 = lax.iota(jnp.int32, LANES) * V
    jbase1 = jbase0 + LANES * V

    def build(g, carry):
        col = g * LANES
        ivec = idx_v[pl.ds(col, LANES)]
        for j in range(LANES):
            i_b = ivec[j]
            widx_v[pl.ds((col + j) * D, LANES)] = jbase0 + i_b
            widx_v[pl.ds((col + j) * D + LANES, LANES)] = jbase1 + i_b
        return carry

    lax.fori_loop(0, BPW // LANES, build, 0, unroll=1)

    def fire(d, carry):
        off = d * GCHUNK
        pltpu.async_copy(
            tbl_hbm.at[widx_v.at[pl.ds(off, GCHUNK)]],
            buf.at[pl.ds(off, GCHUNK)],
            sem,
        )
        return carry

    lax.fori_loop(0, NDESC, fire, 0, unroll=2)
    pltpu.make_async_copy(tbl_hbm.at[pl.ds(0, BPW * D)], buf, sem).wait()

    def row_fn(r, carry):
        v0 = buf[pl.ds(r * D, LANES)]
        v1 = buf[pl.ds(r * D + LANES, LANES)]
        t = _hsum_all_lanes(v0 * v0 + v1 * v1)
        inv = _rsqrt_newton(t)
        buf[pl.ds(r * D, LANES)] = v0 * inv
        buf[pl.ds(r * D + LANES, LANES)] = v1 * inv
        return carry

    lax.fori_loop(0, BPW, row_fn, 0, unroll=2)

    pltpu.sync_copy(buf, out_hbm.at[pl.ds(base * D, BPW * D)])


def kernel(class_indices, embedding_weight):
    if class_indices.ndim > 1:
        class_indices = class_indices.squeeze(-1)
    idx = class_indices.astype(jnp.int32)
    tbl_flat = embedding_weight.T.reshape(-1)

    mesh = plsc.VectorSubcoreMesh(
        core_axis_name="c",
        subcore_axis_name="s",
        num_cores=NUM_CORES,
        num_subcores=NUM_SUBCORES,
    )
    run = pl.kernel(
        _sc_body,
        out_type=jax.ShapeDtypeStruct((B * D,), jnp.float32),
        mesh=mesh,
        scratch_types=[
            pltpu.VMEM((BPW,), jnp.int32),
            pltpu.VMEM((BPW * D,), jnp.int32),
            pltpu.VMEM((BPW * D,), jnp.float32),
            pltpu.SemaphoreType.DMA,
        ],
        compiler_params=pltpu.CompilerParams(use_tc_tiling_on_sc=False),
    )
    out_flat = run(idx, tbl_flat)
    return out_flat.reshape(B, D)


# speedup vs baseline: 4.0575x; 4.0575x over previous
import jax
import jax.numpy as jnp
from jax import lax
from jax.experimental import pallas as pl
from jax.experimental.pallas import tpu as pltpu
from jax.experimental.pallas import tpu_sc as plsc

B = 16384
D = 32
NUM_CORES = 2
NUM_SUBCORES = 16
LANES = 16
NW = NUM_CORES * NUM_SUBCORES
BPW = B // NW
CHUNK = 128
NCHUNK = BPW // CHUNK


def _rsqrt_newton(t):
    i = lax.bitcast_convert_type(t, jnp.int32)
    y = lax.bitcast_convert_type(
        jnp.int32(0x5F3759DF) - lax.shift_right_logical(i, 1), jnp.float32
    )
    ht = t * jnp.float32(0.5)
    for _ in range(3):
        y = y * (jnp.float32(1.5) - ht * y * y)
    return y


def _hsum_all_lanes(v):
    lane = lax.iota(jnp.int32, LANES)
    for k in (8, 4, 2, 1):
        rot = lax.bitwise_and(lane + k, LANES - 1)
        v = v + jnp.take_along_axis(v, rot, axis=0)
    return v


def _sc_body(idx_hbm, tbl_hbm, out_hbm, idx_v, rows_bf, rows_f, sem):
    wid = lax.axis_index("s") * NUM_CORES + lax.axis_index("c")
    base = wid * BPW

    for j in range(NCHUNK):
        pltpu.sync_copy(idx_hbm.at[pl.ds(base + j * CHUNK, CHUNK)], idx_v.at[j])

    copies = [
        pltpu.async_copy(
            tbl_hbm.at[idx_v.at[j]],
            rows_bf.at[pl.ds(j * CHUNK, CHUNK)],
            sem,
        )
        for j in range(NCHUNK)
    ]
    for c in copies:
        c.wait()

    iota = lax.iota(jnp.int32, LANES)
    even = iota * 2
    odd = even + 1

    def row_fn(r, carry):
        row = rows_bf[r, pl.ds(0, D)]
        a, b = plsc.unpack(row, format=plsc.PackFormat.INTERLEAVED)
        t = _hsum_all_lanes(a * a + b * b)
        inv = _rsqrt_newton(t)
        rvec = jnp.full((LANES,), r, jnp.int32)
        plsc.store_scatter(rows_f, [rvec, even], a * inv)
        plsc.store_scatter(rows_f, [rvec, odd], b * inv)
        return carry

    lax.fori_loop(0, BPW, row_fn, 0, unroll=2)

    pltpu.sync_copy(rows_f, out_hbm.at[pl.ds(base, BPW)])


def kernel(class_indices, embedding_weight):
    if class_indices.ndim > 1:
        class_indices = class_indices.squeeze(-1)
    idx = class_indices.astype(jnp.int32)
    tbl_bf = embedding_weight.astype(jnp.bfloat16)

    mesh = plsc.VectorSubcoreMesh(
        core_axis_name="c",
        subcore_axis_name="s",
        num_cores=NUM_CORES,
        num_subcores=NUM_SUBCORES,
    )
    run = pl.kernel(
        _sc_body,
        out_type=jax.ShapeDtypeStruct((B, D), jnp.float32),
        mesh=mesh,
        scratch_types=[
            pltpu.VMEM((NCHUNK, CHUNK), jnp.int32),
            pltpu.VMEM((BPW, D), jnp.bfloat16),
            pltpu.VMEM((BPW, D), jnp.float32),
            pltpu.SemaphoreType.DMA,
        ],
        compiler_params=pltpu.CompilerParams(
            use_tc_tiling_on_sc=False, needs_layout_passes=False
        ),
    )
    return run(idx, tbl_bf)


# speedup vs baseline: 4.7657x; 1.1745x over previous
import jax
import jax.numpy as jnp
from jax import lax
from jax.experimental import pallas as pl
from jax.experimental.pallas import tpu as pltpu
from jax.experimental.pallas import tpu_sc as plsc

B = 16384
D = 32
NUM_CORES = 2
NUM_SUBCORES = 16
LANES = 16
NW = NUM_CORES * NUM_SUBCORES
BPW = B // NW
CHUNK = 128
NCHUNK = BPW // CHUNK
SROW = 128


def _rsqrt_newton(t):
    i = lax.bitcast_convert_type(t, jnp.int32)
    y = lax.bitcast_convert_type(
        jnp.int32(0x5F3759DF) - lax.shift_right_logical(i, 1), jnp.float32
    )
    ht = t * jnp.float32(0.5)
    for _ in range(3):
        y = y * (jnp.float32(1.5) - ht * y * y)
    return y


def _hsum_all_lanes(v):
    lane = lax.iota(jnp.int32, LANES)
    for k in (8, 4, 2, 1):
        rot = lax.bitwise_and(lane + k, LANES - 1)
        v = v + jnp.take_along_axis(v, rot, axis=0)
    return v


def _sc_body(idx_hbm, tbl_hbm, out_hbm, idx_v, qidx_v, rows_s, rows_f, sem):
    wid = lax.axis_index("s") * NUM_CORES + lax.axis_index("c")
    base = wid * BPW

    for j in range(NCHUNK):
        pltpu.sync_copy(idx_hbm.at[pl.ds(base + j * CHUNK, CHUNK)], idx_v.at[j])

    def qfill(g, carry):
        for j in range(NCHUNK):
            v = idx_v[j, pl.ds(g * LANES, LANES)]
            qidx_v[j, pl.ds(g * LANES, LANES)] = lax.shift_right_logical(v, 2)
        return carry

    lax.fori_loop(0, CHUNK // LANES, qfill, 0, unroll=1)

    copies = [
        pltpu.async_copy(
            tbl_hbm.at[qidx_v.at[j]],
            rows_s.at[pl.ds(j * CHUNK, CHUNK)],
            sem,
        )
        for j in range(NCHUNK)
    ]
    for c in copies:
        c.wait()

    for ch in range(NCHUNK):

        def row_fn(g, carry, ch=ch):
            col = g * LANES
            ivec = idx_v[ch, pl.ds(col, LANES)]
            for j in range(LANES):
                r = ch * CHUNK + col + j
                q = lax.bitwise_and(ivec[j], 3) * D
                v0 = rows_s[r, pl.ds(q, LANES)]
                v1 = rows_s[r, pl.ds(q + LANES, LANES)]
                t = _hsum_all_lanes(v0 * v0 + v1 * v1)
                inv = _rsqrt_newton(t)
                rows_f[r, pl.ds(0, LANES)] = v0 * inv
                rows_f[r, pl.ds(LANES, LANES)] = v1 * inv
            return carry

        lax.fori_loop(0, CHUNK // LANES, row_fn, 0, unroll=1)

    pltpu.sync_copy(rows_f, out_hbm.at[pl.ds(base, BPW)])


def kernel(class_indices, embedding_weight):
    if class_indices.ndim > 1:
        class_indices = class_indices.squeeze(-1)
    idx = class_indices.astype(jnp.int32)
    tbl_s = embedding_weight.reshape(250000, SROW)

    mesh = plsc.VectorSubcoreMesh(
        core_axis_name="c",
        subcore_axis_name="s",
        num_cores=NUM_CORES,
        num_subcores=NUM_SUBCORES,
    )
    run = pl.kernel(
        _sc_body,
        out_type=jax.ShapeDtypeStruct((B, D), jnp.float32),
        mesh=mesh,
        scratch_types=[
            pltpu.VMEM((NCHUNK, CHUNK), jnp.int32),
            pltpu.VMEM((NCHUNK, CHUNK), jnp.int32),
            pltpu.VMEM((BPW, SROW), jnp.float32),
            pltpu.VMEM((BPW, D), jnp.float32),
            pltpu.SemaphoreType.DMA,
        ],
        compiler_params=pltpu.CompilerParams(use_tc_tiling_on_sc=False),
    )
    return run(idx, tbl_s)


# speedup vs baseline: 4.8710x; 1.0221x over previous
import jax
import jax.numpy as jnp
from jax import lax
from jax.experimental import pallas as pl
from jax.experimental.pallas import tpu as pltpu
from jax.experimental.pallas import tpu_sc as plsc

B = 16384
D = 32
NUM_CORES = 2
NUM_SUBCORES = 16
LANES = 16
NW = NUM_CORES * NUM_SUBCORES
BPW = B // NW
CHUNK = 128
NCHUNK = BPW // CHUNK


def _rsqrt_newton(t):
    i = lax.bitcast_convert_type(t, jnp.int32)
    y = lax.bitcast_convert_type(
        jnp.int32(0x5F3759DF) - lax.shift_right_logical(i, 1), jnp.float32
    )
    ht = t * jnp.float32(0.5)
    for _ in range(3):
        y = y * (jnp.float32(1.5) - ht * y * y)
    return y


def _hsum_all_lanes(v):
    lane = lax.iota(jnp.int32, LANES)
    for k in (8, 4, 2, 1):
        rot = lax.bitwise_and(lane + k, LANES - 1)
        v = v + jnp.take_along_axis(v, rot, axis=0)
    return v


def _sc_body(idx_hbm, table_hbm, out_hbm, idx_v, rows_v, sem):
    wid = lax.axis_index("s") * NUM_CORES + lax.axis_index("c")
    base = wid * BPW

    for j in range(NCHUNK):
        pltpu.sync_copy(idx_hbm.at[pl.ds(base + j * CHUNK, CHUNK)], idx_v.at[j])

    copies = [
        pltpu.async_copy(
            table_hbm.at[idx_v.at[j]],
            rows_v.at[pl.ds(j * CHUNK, CHUNK)],
            sem,
        )
        for j in range(NCHUNK)
    ]
    for c in copies:
        c.wait()

    def row_fn(r, carry):
        v0 = rows_v[r, pl.ds(0, LANES)]
        v1 = rows_v[r, pl.ds(LANES, LANES)]
        t = _hsum_all_lanes(v0 * v0 + v1 * v1)
        inv = _rsqrt_newton(t)
        rows_v[r, pl.ds(0, LANES)] = v0 * inv
        rows_v[r, pl.ds(LANES, LANES)] = v1 * inv
        return carry

    lax.fori_loop(0, BPW, row_fn, 0, unroll=2)

    pltpu.sync_copy(rows_v, out_hbm.at[pl.ds(base, BPW)])


def kernel(class_indices, embedding_weight):
    if class_indices.ndim > 1:
        class_indices = class_indices.squeeze(-1)
    idx = class_indices.astype(jnp.int32)

    mesh = plsc.VectorSubcoreMesh(
        core_axis_name="c",
        subcore_axis_name="s",
        num_cores=NUM_CORES,
        num_subcores=NUM_SUBCORES,
    )
    run = pl.kernel(
        _sc_body,
        out_type=jax.ShapeDtypeStruct((B, D), jnp.float32),
        mesh=mesh,
        scratch_types=[
            pltpu.VMEM((NCHUNK, CHUNK), jnp.int32),
            pltpu.VMEM((BPW, D), jnp.float32),
            pltpu.SemaphoreType.DMA,
        ],
        compiler_params=pltpu.CompilerParams(use_tc_tiling_on_sc=False),
    )
    return run(idx, embedding_weight)
